# baseline (speedup 1.0000x reference)
import functools

import jax
import jax.numpy as jnp
from jax import lax
from jax.experimental import pallas as pl
from jax.experimental.pallas import tpu as pltpu
from jax.experimental.pallas import tpu_sc as plsc

F = 26
V = 100000
D = 16
B = 16384
NC, NS, L = 2, 16, 16
SUB = B // NS
HS = V // 2
NP = 2 * F
IST = 2048
NST = B // IST
UNR = 8
CB = 128
NCB = SUB // CB


@functools.partial(
    pl.kernel,
    mesh=plsc.VectorSubcoreMesh(core_axis_name="c", subcore_axis_name="s"),
    out_type=(
        jax.ShapeDtypeStruct((B,), jnp.float32),
        jax.ShapeDtypeStruct((B,), jnp.float32),
        jax.ShapeDtypeStruct((NC, NS, B), jnp.float32),
    ),
    scratch_types=[
        pltpu.VMEM((2 * HS,), jnp.float32),
        pltpu.VMEM((B,), jnp.float32),
        pltpu.VMEM((2 * IST,), jnp.int32),
        pltpu.VMEM((F, CB), jnp.int32),
        pltpu.VMEM((F * CB,), jnp.int32),
        pltpu.VMEM((F * CB,), jnp.float32),
        pltpu.VMEM((CB,), jnp.float32),
        pltpu.SemaphoreType.DMA,
        pltpu.SemaphoreType.DMA,
        pltpu.SemaphoreType.DMA,
    ],
    compiler_params=pltpu.CompilerParams(
        needs_layout_passes=False, use_tc_tiling_on_sc=False
    ),
)
def _fm_sc(pos_t, neg_t, wft, wl, out_p, out_n, part, slab_v, acc_v, idxa_v,
           rawb_v, gidx_v, lin_v, outb_v, sems, semi, seml):
    c = lax.axis_index("c")
    s = lax.axis_index("s")

    def phase_a(src):
        def start_slab(k):
            f2 = k // 2
            h2 = k % 2
            return pltpu.async_copy(
                wft.at[f2 * D + s, pl.ds(h2 * HS, HS)],
                slab_v.at[pl.ds(h2 * HS, HS)], sems)

        start_slab(0)

        def pass_body(k, carry):
            h = k % 2
            f = k // 2
            lo = h * HS
            hi = lo + HS
            pltpu.make_async_copy(
                wft.at[0, pl.ds(0, HS)], slab_v.at[pl.ds(0, HS)], sems
            ).wait()

            @pl.when(k < NP - 1)
            def _():
                start_slab(k + 1)

            pltpu.sync_copy(src.at[f, pl.ds(0, IST)],
                            idxa_v.at[pl.ds(0, IST)])
            for st in range(NST):
                sb = (st % 2) * IST
                if st + 1 < NST:
                    nb = ((st + 1) % 2) * IST
                    cpi = pltpu.async_copy(
                        src.at[f, pl.ds((st + 1) * IST, IST)],
                        idxa_v.at[pl.ds(nb, IST)], semi)

                def vec_body(v, carry2):
                    for u in range(UNR):
                        off = st * IST + (v * UNR + u) * L
                        idx16 = idxa_v[pl.ds(sb + (v * UNR + u) * L, L)]
                        m = (idx16 >= lo) & (idx16 < hi)
                        val = plsc.load_gather(slab_v, [idx16], mask=m)
                        val = jnp.where(m, val, 0.0)
                        acc_v[pl.ds(off, L)] = acc_v[pl.ds(off, L)] + val
                    return carry2

                lax.fori_loop(0, IST // (L * UNR), vec_body, 0)
                if st + 1 < NST:
                    cpi.wait()
            return carry

        pltpu.make_async_copy(
            wft.at[0, pl.ds(0, HS)], slab_v.at[pl.ds(0, HS)], sems).wait()
        start_slab(1)
        pltpu.sync_copy(src.at[0, pl.ds(0, IST)], idxa_v.at[pl.ds(0, IST)])
        for st in range(NST):
            sb = (st % 2) * IST
            if st + 1 < NST:
                nb = ((st + 1) % 2) * IST
                cpi0 = pltpu.async_copy(
                    src.at[0, pl.ds((st + 1) * IST, IST)],
                    idxa_v.at[pl.ds(nb, IST)], semi)

            def vec0_body(v, carry2):
                for u in range(UNR):
                    off = st * IST + (v * UNR + u) * L
                    idx16 = idxa_v[pl.ds(sb + (v * UNR + u) * L, L)]
                    m = idx16 < HS
                    val = plsc.load_gather(slab_v, [idx16], mask=m)
                    acc_v[pl.ds(off, L)] = jnp.where(m, val, 0.0)
                return carry2

            lax.fori_loop(0, IST // (L * UNR), vec0_body, 0)
            if st + 1 < NST:
                cpi0.wait()
        lax.fori_loop(1, NP, pass_body, 0)
        pltpu.sync_copy(acc_v, part.at[c, s])
        plsc.subcore_barrier()

    def phase_b(src, dst):
        for d in range(NS):
            pltpu.sync_copy(part.at[c, d, pl.ds(s * SUB, SUB)],
                            acc_v.at[pl.ds(d * SUB, SUB)])

        def chunk_body(cb, carry):
            b0 = s * SUB + cb * CB
            pltpu.sync_copy(src.at[:, pl.ds(b0, CB)], rawb_v)
            for f in range(F):
                for s2 in range(CB // L):
                    v16 = rawb_v[f, pl.ds(s2 * L, L)] + (f * V)
                    gidx_v[pl.ds(f * CB + s2 * L, L)] = v16
            pltpu.async_copy(wl.at[gidx_v], lin_v, seml).wait()
            for g in range(CB // L):
                o = cb * CB + g * L
                t = acc_v[pl.ds(o, L)]
                s_vec = t
                q_vec = t * t
                for d in range(1, NS):
                    t = acc_v[pl.ds(d * SUB + o, L)]
                    s_vec = s_vec + t
                    q_vec = q_vec + t * t
                pv = lin_v[pl.ds(g * L, L)]
                for f in range(1, F):
                    pv = pv + lin_v[pl.ds(f * CB + g * L, L)]
                outb_v[pl.ds(g * L, L)] = pv + 0.5 * (s_vec * s_vec - q_vec)
            pltpu.sync_copy(outb_v, dst.at[pl.ds(b0, CB)])
            return carry

        lax.fori_loop(0, NCB, chunk_body, 0)

    @pl.when(c == 0)
    def _():
        phase_a(pos_t)
        phase_b(pos_t, out_p)

    @pl.when(c == 1)
    def _():
        phase_a(neg_t)
        phase_b(neg_t, out_n)


def _fm_host(pos_batch, neg_batch, W_linear, W_factor):
    pos_t = pos_batch.T
    neg_t = neg_batch.T
    wft = jnp.swapaxes(W_factor, 1, 2).reshape(F * D, V)
    wl = W_linear.reshape(F * V)
    preds_p, preds_n, _ = _fm_sc(pos_t, neg_t, wft, wl)
    return preds_p, preds_n


def kernel(pos_batch, neg_batch, W_linear, W_factor):
    preds_p, preds_n = _fm_host(pos_batch, neg_batch, W_linear, W_factor)
    pos_preds = preds_p[:, None]
    neg_preds = preds_n[:, None]
    l2 = jnp.zeros((1,), jnp.float32)
    return (pos_preds, neg_preds, l2)


# speedup vs baseline: 1.1117x; 1.1117x over previous
import functools

import jax
import jax.numpy as jnp
from jax import lax
from jax.experimental import pallas as pl
from jax.experimental.pallas import tpu as pltpu
from jax.experimental.pallas import tpu_sc as plsc

F = 26
V = 100000
D = 16
B = 16384
NC, NS, L = 2, 16, 16
SUB = B // NS
IST = 2048
NST = B // IST
CB = 128
NCB = SUB // CB


@functools.partial(
    pl.kernel,
    mesh=plsc.VectorSubcoreMesh(core_axis_name="c", subcore_axis_name="s"),
    out_type=(
        jax.ShapeDtypeStruct((B,), jnp.float32),
        jax.ShapeDtypeStruct((B,), jnp.float32),
        jax.ShapeDtypeStruct((NC, NS, B), jnp.float32),
    ),
    scratch_types=[
        pltpu.VMEM((V,), jnp.float32),
        pltpu.VMEM((B,), jnp.float32),
        pltpu.VMEM((2 * IST,), jnp.int32),
        pltpu.VMEM((F, CB), jnp.int32),
        pltpu.VMEM((F * CB,), jnp.int32),
        pltpu.VMEM((F * CB,), jnp.float32),
        pltpu.VMEM((CB,), jnp.float32),
        pltpu.SemaphoreType.DMA,
        pltpu.SemaphoreType.DMA,
    ],
    compiler_params=pltpu.CompilerParams(
        needs_layout_passes=False, use_tc_tiling_on_sc=False
    ),
)
def _fm_sc(pos_t, neg_t, wft, wl, out_p, out_n, part, slab_v, acc_v, idxa_v,
           rawb_v, gidx_v, lin_v, outb_v, semi, seml):
    c = lax.axis_index("c")
    s = lax.axis_index("s")

    UNR = 8

    def phase_a(src):
        def zero_body(i, carry):
            acc_v[pl.ds(i * L, L)] = jnp.zeros((L,), jnp.float32)
            return carry

        lax.fori_loop(0, B // L, zero_body, 0)

        def field_body(f, carry):
            pltpu.sync_copy(wft.at[f * D + s], slab_v)
            pltpu.sync_copy(src.at[f, pl.ds(0, IST)], idxa_v.at[pl.ds(0, IST)])
            for st in range(NST):
                sb = (st % 2) * IST
                if st + 1 < NST:
                    cpi = pltpu.async_copy(
                        src.at[f, pl.ds((st + 1) * IST, IST)],
                        idxa_v.at[pl.ds(((st + 1) % 2) * IST, IST)], semi)

                def vec_body(v, carry2):
                    for u in range(UNR):
                        off = st * IST + (v * UNR + u) * L
                        idx16 = idxa_v[pl.ds(sb + (v * UNR + u) * L, L)]
                        val = plsc.load_gather(slab_v, [idx16])
                        acc_v[pl.ds(off, L)] = acc_v[pl.ds(off, L)] + val
                    return carry2

                lax.fori_loop(0, IST // (L * UNR), vec_body, 0)
                if st + 1 < NST:
                    cpi.wait()
            return carry

        lax.fori_loop(0, F, field_body, 0)
        pltpu.sync_copy(acc_v, part.at[c, s])
        plsc.subcore_barrier()

    def phase_b(src, dst):
        for d in range(NS):
            pltpu.sync_copy(part.at[c, d, pl.ds(s * SUB, SUB)],
                            acc_v.at[pl.ds(d * SUB, SUB)])

        def chunk_body(cb, carry):
            b0 = s * SUB + cb * CB
            pltpu.sync_copy(src.at[:, pl.ds(b0, CB)], rawb_v)
            for f in range(F):
                for s2 in range(CB // L):
                    v16 = rawb_v[f, pl.ds(s2 * L, L)] + (f * V)
                    gidx_v[pl.ds(f * CB + s2 * L, L)] = v16
            pltpu.async_copy(wl.at[gidx_v], lin_v, seml).wait()
            for g in range(CB // L):
                o = cb * CB + g * L
                t = acc_v[pl.ds(o, L)]
                s_vec = t
                q_vec = t * t
                for d in range(1, NS):
                    t = acc_v[pl.ds(d * SUB + o, L)]
                    s_vec = s_vec + t
                    q_vec = q_vec + t * t
                pv = lin_v[pl.ds(g * L, L)]
                for f in range(1, F):
                    pv = pv + lin_v[pl.ds(f * CB + g * L, L)]
                outb_v[pl.ds(g * L, L)] = pv + 0.5 * (s_vec * s_vec - q_vec)
            pltpu.sync_copy(outb_v, dst.at[pl.ds(b0, CB)])
            return carry

        lax.fori_loop(0, NCB, chunk_body, 0)

    @pl.when(c == 0)
    def _():
        phase_a(pos_t)
        phase_b(pos_t, out_p)

    @pl.when(c == 1)
    def _():
        phase_a(neg_t)
        phase_b(neg_t, out_n)


def _fm_host(pos_batch, neg_batch, W_linear, W_factor):
    pos_t = pos_batch.T
    neg_t = neg_batch.T
    wft = jnp.swapaxes(W_factor, 1, 2).reshape(F * D, V)
    wl = W_linear.reshape(F * V)
    preds_p, preds_n, _ = _fm_sc(pos_t, neg_t, wft, wl)
    return preds_p, preds_n


def kernel(pos_batch, neg_batch, W_linear, W_factor):
    preds_p, preds_n = _fm_host(pos_batch, neg_batch, W_linear, W_factor)
    pos_preds = preds_p[:, None]
    neg_preds = preds_n[:, None]
    l2 = jnp.zeros((1,), jnp.float32)
    return (pos_preds, neg_preds, l2)


# speedup vs baseline: 1.1341x; 1.0202x over previous
import functools

import jax
import jax.numpy as jnp
from jax import lax
from jax.experimental import pallas as pl
from jax.experimental.pallas import tpu as pltpu
from jax.experimental.pallas import tpu_sc as plsc

F = 26
V = 100000
D = 16
B = 16384
NC, NS, L = 2, 16, 16
SUB = B // NS
IST = 2048
NST = B // IST
CB = 128
NCB = SUB // CB


@functools.partial(
    pl.kernel,
    mesh=plsc.VectorSubcoreMesh(core_axis_name="c", subcore_axis_name="s"),
    out_type=(
        jax.ShapeDtypeStruct((B,), jnp.float32),
        jax.ShapeDtypeStruct((B,), jnp.float32),
        jax.ShapeDtypeStruct((NC, NS, B), jnp.float32),
    ),
    scratch_types=[
        pltpu.VMEM((V,), jnp.float32),
        pltpu.VMEM((B,), jnp.float32),
        pltpu.VMEM((2 * IST,), jnp.int32),
        pltpu.VMEM((F, CB), jnp.int32),
        pltpu.VMEM((F * CB,), jnp.int32),
        pltpu.VMEM((F * CB,), jnp.float32),
        pltpu.VMEM((CB,), jnp.float32),
        pltpu.SemaphoreType.DMA,
        pltpu.SemaphoreType.DMA,
        pltpu.SemaphoreType.DMA,
    ],
    compiler_params=pltpu.CompilerParams(
        needs_layout_passes=False, use_tc_tiling_on_sc=False
    ),
)
def _fm_sc(pos_t, neg_t, wft, wl, out_p, out_n, part, slab_v, acc_v, idxa_v,
           rawb_v, gidx_v, lin_v, outb_v, sems, semi, seml):
    c = lax.axis_index("c")
    s = lax.axis_index("s")

    UNR = 8

    def phase_a(src):
        def zero_body(i, carry):
            acc_v[pl.ds(i * L, L)] = jnp.zeros((L,), jnp.float32)
            return carry

        lax.fori_loop(0, B // L, zero_body, 0)

        QS = V // 4

        def field_body(f, carry):
            cps = [
                pltpu.async_copy(
                    wft.at[f * D + s, pl.ds(q * QS, QS)],
                    slab_v.at[pl.ds(q * QS, QS)], sems)
                for q in range(4)
            ]
            pltpu.sync_copy(src.at[f, pl.ds(0, IST)], idxa_v.at[pl.ds(0, IST)])
            for cp in cps:
                cp.wait()
            for st in range(NST):
                sb = (st % 2) * IST
                if st + 1 < NST:
                    cpi = pltpu.async_copy(
                        src.at[f, pl.ds((st + 1) * IST, IST)],
                        idxa_v.at[pl.ds(((st + 1) % 2) * IST, IST)], semi)

                def vec_body(v, carry2):
                    for u in range(UNR):
                        off = st * IST + (v * UNR + u) * L
                        idx16 = idxa_v[pl.ds(sb + (v * UNR + u) * L, L)]
                        val = plsc.load_gather(slab_v, [idx16])
                        acc_v[pl.ds(off, L)] = acc_v[pl.ds(off, L)] + val
                    return carry2

                lax.fori_loop(0, IST // (L * UNR), vec_body, 0)
                if st + 1 < NST:
                    cpi.wait()
            return carry

        lax.fori_loop(0, F, field_body, 0)
        pltpu.sync_copy(acc_v, part.at[c, s])
        plsc.subcore_barrier()

    def phase_b(src, dst):
        for d in range(NS):
            pltpu.sync_copy(part.at[c, d, pl.ds(s * SUB, SUB)],
                            acc_v.at[pl.ds(d * SUB, SUB)])

        def chunk_body(cb, carry):
            b0 = s * SUB + cb * CB
            pltpu.sync_copy(src.at[:, pl.ds(b0, CB)], rawb_v)
            for f in range(F):
                for s2 in range(CB // L):
                    v16 = rawb_v[f, pl.ds(s2 * L, L)] + (f * V)
                    gidx_v[pl.ds(f * CB + s2 * L, L)] = v16
            pltpu.async_copy(wl.at[gidx_v], lin_v, seml).wait()
            for g in range(CB // L):
                o = cb * CB + g * L
                t = acc_v[pl.ds(o, L)]
                s_vec = t
                q_vec = t * t
                for d in range(1, NS):
                    t = acc_v[pl.ds(d * SUB + o, L)]
                    s_vec = s_vec + t
                    q_vec = q_vec + t * t
                pv = lin_v[pl.ds(g * L, L)]
                for f in range(1, F):
                    pv = pv + lin_v[pl.ds(f * CB + g * L, L)]
                outb_v[pl.ds(g * L, L)] = pv + 0.5 * (s_vec * s_vec - q_vec)
            pltpu.sync_copy(outb_v, dst.at[pl.ds(b0, CB)])
            return carry

        lax.fori_loop(0, NCB, chunk_body, 0)

    @pl.when(c == 0)
    def _():
        phase_a(pos_t)
        phase_b(pos_t, out_p)

    @pl.when(c == 1)
    def _():
        phase_a(neg_t)
        phase_b(neg_t, out_n)


def _fm_host(pos_batch, neg_batch, W_linear, W_factor):
    pos_t = pos_batch.T
    neg_t = neg_batch.T
    wft = jnp.swapaxes(W_factor, 1, 2).reshape(F * D, V)
    wl = W_linear.reshape(F * V)
    preds_p, preds_n, _ = _fm_sc(pos_t, neg_t, wft, wl)
    return preds_p, preds_n


def kernel(pos_batch, neg_batch, W_linear, W_factor):
    preds_p, preds_n = _fm_host(pos_batch, neg_batch, W_linear, W_factor)
    pos_preds = preds_p[:, None]
    neg_preds = preds_n[:, None]
    l2 = jnp.zeros((1,), jnp.float32)
    return (pos_preds, neg_preds, l2)
